# Initial kernel scaffold; baseline (speedup 1.0000x reference)
import jax
import jax.numpy as jnp
from jax.experimental import pallas as pl


def kernel(x, edge_index, edge_attr, u, batch, W1, b1, gamma, beta, W2, b2):
    raise NotImplementedError("write your pallas kernel here")


# speedup vs baseline: 4.5475x; 4.5475x over previous
import functools

import jax
import jax.numpy as jnp
from jax import lax
from jax.experimental import pallas as pl
from jax.experimental.pallas import tpu as pltpu
from jax.experimental.pallas import tpu_sc as plsc

_NC = 2
_NS = 16
_NW = _NC * _NS


def _make_sc_segsum(n_nodes, n_edges, d_edge, chunk):
    epw = n_edges // _NW
    nchunk = epw // chunk
    rows_per_tile = n_nodes // _NS

    mesh = plsc.VectorSubcoreMesh(core_axis_name="c", subcore_axis_name="s")

    @functools.partial(
        pl.kernel,
        out_type=(
            jax.ShapeDtypeStruct((_NC, n_nodes, d_edge), jnp.float32),
            jax.ShapeDtypeStruct((_NC, n_nodes, d_edge), jnp.float32),
        ),
        mesh=mesh,
        compiler_params=pltpu.CompilerParams(use_tc_tiling_on_sc=False),
        scratch_types=[
            pltpu.VMEM((nchunk, chunk), jnp.int32),
            pltpu.VMEM((chunk, d_edge), jnp.float32),
            pltpu.VMEM((chunk, d_edge), jnp.float32),
            pltpu.VMEM((rows_per_tile, d_edge), jnp.float32),
            pltpu.VMEM_SHARED((n_nodes, d_edge), jnp.float32),
            pltpu.VMEM_SHARED((n_nodes, d_edge), jnp.float32),
        ],
    )
    def sc_segsum(col_hbm, edge_hbm, sums_out, cnt_out,
                  idx_v, val_v, ones_v, zero_v, sums_sh, cnt_sh):
        cid = lax.axis_index("c")
        sid = lax.axis_index("s")
        wid = sid * _NC + cid

        def fill(ref, value, n):
            def body(i, _):
                ref[i, :] = jnp.full((d_edge,), value, jnp.float32)
                return 0
            lax.fori_loop(0, n, body, 0)

        fill(zero_v, 0.0, rows_per_tile)
        fill(ones_v, 1.0, chunk)

        pltpu.sync_copy(zero_v, sums_sh.at[pl.ds(sid * rows_per_tile, rows_per_tile)])
        pltpu.sync_copy(zero_v, cnt_sh.at[pl.ds(sid * rows_per_tile, rows_per_tile)])
        plsc.subcore_barrier()

        pltpu.sync_copy(col_hbm.at[wid], idx_v)

        def chunk_body(j, _):
            pltpu.sync_copy(edge_hbm.at[wid, j], val_v)
            pltpu.sync_copy(val_v, sums_sh.at[idx_v.at[j]], add=True)
            pltpu.sync_copy(ones_v, cnt_sh.at[idx_v.at[j]], add=True)
            return 0
        lax.fori_loop(0, nchunk, chunk_body, 0)
        plsc.subcore_barrier()

        @pl.when(sid == 0)
        def _():
            pltpu.sync_copy(sums_sh, sums_out.at[cid])
            pltpu.sync_copy(cnt_sh, cnt_out.at[cid])

    return sc_segsum


def _tc_dense(x, s0, s1, c0, c1, batch2d, u, w1x, w1e, w1u, b1, gamma, beta, w2, b2):
    n, d_in = x.shape
    n_graphs = u.shape[0]
    hidden = w1x.shape[1]
    d_out = w2.shape[1]

    def body(x_ref, s0_ref, s1_ref, c0_ref, c1_ref, b_ref, u_ref,
             w1x_ref, w1e_ref, w1u_ref, b1_ref, g_ref, be_ref, w2_ref, b2_ref,
             out_ref):
        sums = s0_ref[...] + s1_ref[...]
        cnt = c0_ref[...] + c1_ref[...]
        e_aggr = sums / jnp.maximum(cnt, 1.0)
        oh = (b_ref[...] == lax.broadcasted_iota(jnp.int32, (n, n_graphs), 1))
        ub = jnp.dot(oh.astype(jnp.float32), u_ref[...],
                     preferred_element_type=jnp.float32)
        h = (jnp.dot(x_ref[...], w1x_ref[...], preferred_element_type=jnp.float32)
             + jnp.dot(e_aggr, w1e_ref[...], preferred_element_type=jnp.float32)
             + jnp.dot(ub, w1u_ref[...], preferred_element_type=jnp.float32)
             + b1_ref[...])
        mean = jnp.mean(h, axis=0, keepdims=True)
        var = jnp.mean((h - mean) ** 2, axis=0, keepdims=True)
        hn = (h - mean) / jnp.sqrt(var + 1e-5) * g_ref[...] + be_ref[...]
        hr = jnp.maximum(hn, 0.0)
        out_ref[...] = (jnp.dot(hr, w2_ref[...], preferred_element_type=jnp.float32)
                        + b2_ref[...])

    return pl.pallas_call(
        body,
        out_shape=jax.ShapeDtypeStruct((n, d_out), jnp.float32),
    )(x, s0, s1, c0, c1, batch2d, u, w1x, w1e, w1u, b1, gamma, beta, w2, b2)


def kernel(x, edge_index, edge_attr, u, batch, W1, b1, gamma, beta, W2, b2):
    n_nodes, d_node = x.shape
    n_edges, d_edge = edge_attr.shape
    d_graph = u.shape[1]
    hidden = W1.shape[0]

    chunk = 80
    col = edge_index[1].astype(jnp.int32)
    epw = n_edges // _NW
    col3 = col.reshape(_NW, epw // chunk, chunk)
    edge4 = edge_attr.reshape(_NW, epw // chunk, chunk, d_edge)

    sc_segsum = _make_sc_segsum(n_nodes, n_edges, d_edge, chunk)
    sums, counts = sc_segsum(col3, edge4)

    w1t = W1.T
    w1x = w1t[:d_node]
    w1e = w1t[d_node:d_node + d_edge]
    w1u = w1t[d_node + d_edge:]

    return _tc_dense(
        x, sums[0], sums[1], counts[0], counts[1],
        batch.astype(jnp.int32).reshape(n_nodes, 1), u,
        w1x, w1e, w1u,
        b1.reshape(1, hidden), gamma.reshape(1, hidden), beta.reshape(1, hidden),
        W2.T, b2.reshape(1, W2.shape[0]),
    )


# speedup vs baseline: 5.4017x; 1.1879x over previous
import functools

import jax
import jax.numpy as jnp
from jax import lax
from jax.experimental import pallas as pl
from jax.experimental.pallas import tpu as pltpu
from jax.experimental.pallas import tpu_sc as plsc

_NC = 2
_NS = 16
_NW = _NC * _NS
_CHUNK = 125
_CPS = 10
_RS = _CHUNK * _CPS


def _make_sc_segsum(n_nodes, n_edges, d_edge):
    epw = n_edges // _NW
    nchunk = epw // _CHUNK
    nsup = nchunk // _CPS
    zrows = n_nodes // _NS
    dw = 2 * d_edge

    mesh = plsc.VectorSubcoreMesh(core_axis_name="c", subcore_axis_name="s")

    @functools.partial(
        pl.kernel,
        out_type=jax.ShapeDtypeStruct((_NC, n_nodes, dw), jnp.float32),
        mesh=mesh,
        compiler_params=pltpu.CompilerParams(use_tc_tiling_on_sc=False),
        scratch_types=[
            pltpu.VMEM((nchunk, _CHUNK), jnp.int32),
            pltpu.VMEM((2, _RS, dw), jnp.float32),
            pltpu.VMEM_SHARED((n_nodes, dw), jnp.float32),
            pltpu.SemaphoreType.DMA,
            pltpu.SemaphoreType.DMA,
        ],
    )
    def sc_segsum(col_hbm, edge_hbm, acc_out,
                  idx_v, val_v, acc_sh, gsem, ssem):
        cid = lax.axis_index("c")
        sid = lax.axis_index("s")
        wid = sid * _NC + cid

        def zfill(i, _):
            val_v[0, i, pl.ds(0, d_edge)] = jnp.zeros((d_edge,), jnp.float32)
            val_v[0, i, pl.ds(d_edge, d_edge)] = jnp.zeros((d_edge,), jnp.float32)
            return 0
        lax.fori_loop(0, zrows, zfill, 0)
        pltpu.sync_copy(val_v.at[0, pl.ds(0, zrows)],
                        acc_sh.at[pl.ds(sid * zrows, zrows)])

        def ofill(i, _):
            val_v[0, i, pl.ds(d_edge, d_edge)] = jnp.ones((d_edge,), jnp.float32)
            val_v[1, i, pl.ds(d_edge, d_edge)] = jnp.ones((d_edge,), jnp.float32)
            return 0
        lax.fori_loop(0, _RS, ofill, 0)

        pltpu.sync_copy(col_hbm.at[wid], idx_v)
        plsc.subcore_barrier()

        def gather(s, b):
            return pltpu.async_copy(
                edge_hbm.at[wid, pl.ds(s * _RS, _RS)],
                val_v.at[b, :, pl.ds(0, d_edge)], gsem)

        def fire_scatters(s, b):
            def body(j, _):
                pltpu.async_copy(
                    val_v.at[b, pl.ds(j * _CHUNK, _CHUNK), :],
                    acc_sh.at[idx_v.at[s * _CPS + j]], ssem, add=True)
                return 0
            lax.fori_loop(0, _CPS, body, 0)

        def drain_scatters(b):
            pltpu.make_async_copy(
                acc_out.at[cid, pl.ds(0, _RS)], val_v.at[b], ssem).wait()

        desc = [None, None]
        desc[0] = gather(0, 0)
        for s in range(nsup):
            b = s % 2
            desc[b].wait()
            if s + 1 < nsup:
                if s >= 1:
                    drain_scatters(1 - b)
                desc[1 - b] = gather(s + 1, 1 - b)
            fire_scatters(s, b)
        drain_scatters(nsup % 2)
        drain_scatters((nsup - 1) % 2)
        plsc.subcore_barrier()

        @pl.when(sid == 0)
        def _():
            pltpu.sync_copy(acc_sh, acc_out.at[cid])

    return sc_segsum


def _tc_dense(x, p0, p1, batch2d, u, w1x, w1e, w1u, b1, gamma, beta, w2, b2):
    n, d_node = x.shape
    n_graphs = u.shape[0]
    d_edge = u.shape[1]
    d_out = w2.shape[1]

    def body(x_ref, p0_ref, p1_ref, b_ref, u_ref,
             w1x_ref, w1e_ref, w1u_ref, b1_ref, g_ref, be_ref, w2_ref, b2_ref,
             out_ref):
        acc = p0_ref[...] + p1_ref[...]
        sums = acc[:, :d_edge]
        cnt = acc[:, d_edge:]
        e_aggr = sums / jnp.maximum(cnt, 1.0)
        oh = (b_ref[...] == lax.broadcasted_iota(jnp.int32, (n, n_graphs), 1))
        ub = jnp.dot(oh.astype(jnp.float32), u_ref[...],
                     preferred_element_type=jnp.float32)
        h = (jnp.dot(x_ref[...], w1x_ref[...], preferred_element_type=jnp.float32)
             + jnp.dot(e_aggr, w1e_ref[...], preferred_element_type=jnp.float32)
             + jnp.dot(ub, w1u_ref[...], preferred_element_type=jnp.float32)
             + b1_ref[...])
        mean = jnp.mean(h, axis=0, keepdims=True)
        var = jnp.mean((h - mean) ** 2, axis=0, keepdims=True)
        hn = (h - mean) / jnp.sqrt(var + 1e-5) * g_ref[...] + be_ref[...]
        hr = jnp.maximum(hn, 0.0)
        out_ref[...] = (jnp.dot(hr, w2_ref[...], preferred_element_type=jnp.float32)
                        + b2_ref[...])

    return pl.pallas_call(
        body,
        out_shape=jax.ShapeDtypeStruct((n, d_out), jnp.float32),
    )(x, p0, p1, batch2d, u, w1x, w1e, w1u, b1, gamma, beta, w2, b2)


def kernel(x, edge_index, edge_attr, u, batch, W1, b1, gamma, beta, W2, b2):
    n_nodes, d_node = x.shape
    n_edges, d_edge = edge_attr.shape
    hidden = W1.shape[0]

    col = edge_index[1].astype(jnp.int32)
    epw = n_edges // _NW
    col3 = col.reshape(_NW, epw // _CHUNK, _CHUNK)
    edge3 = edge_attr.reshape(_NW, epw, d_edge)

    sc_segsum = _make_sc_segsum(n_nodes, n_edges, d_edge)
    acc = sc_segsum(col3, edge3)

    w1t = W1.T
    w1x = w1t[:d_node]
    w1e = w1t[d_node:d_node + d_edge]
    w1u = w1t[d_node + d_edge:]

    return _tc_dense(
        x, acc[0], acc[1],
        batch.astype(jnp.int32).reshape(n_nodes, 1), u,
        w1x, w1e, w1u,
        b1.reshape(1, hidden), gamma.reshape(1, hidden), beta.reshape(1, hidden),
        W2.T, b2.reshape(1, W2.shape[0]),
    )


# speedup vs baseline: 6.3085x; 1.1679x over previous
import functools

import jax
import jax.numpy as jnp
from jax import lax
from jax.experimental import pallas as pl
from jax.experimental.pallas import tpu as pltpu
from jax.experimental.pallas import tpu_sc as plsc

_NC = 2
_NS = 16
_NW = _NC * _NS
_CHUNK = 125
_CPS = 10
_RS = _CHUNK * _CPS
_L = 16


def _make_sc_segsum(n_nodes, n_edges, d_edge):
    epw = n_edges // _NW
    nchunk = epw // _CHUNK
    nsup = nchunk // _CPS
    zrows = n_nodes // _NS
    hrows = epw // _L
    crows = -(-(n_nodes // _L) // _NS) * _NS
    czrows = crows // _NS
    mrg = -(-(n_nodes // _L) // _CHUNK)

    mesh = plsc.VectorSubcoreMesh(core_axis_name="c", subcore_axis_name="s")

    @functools.partial(
        pl.kernel,
        out_type=(
            jax.ShapeDtypeStruct((_NC, n_nodes, d_edge), jnp.float32),
            jax.ShapeDtypeStruct((_NC, crows, _L), jnp.float32),
        ),
        mesh=mesh,
        compiler_params=pltpu.CompilerParams(use_tc_tiling_on_sc=False,
                                             needs_layout_passes=False),
        scratch_types=[
            pltpu.VMEM((nchunk, _CHUNK), jnp.int32),
            pltpu.VMEM((hrows, _L), jnp.int32),
            pltpu.VMEM((mrg, _CHUNK), jnp.int32),
            pltpu.VMEM((2, _RS, d_edge), jnp.float32),
            pltpu.VMEM((n_nodes // _L, _L), jnp.float32),
            pltpu.VMEM_SHARED((n_nodes, d_edge), jnp.float32),
            pltpu.VMEM_SHARED((crows, _L), jnp.float32),
            pltpu.SemaphoreType.DMA,
            pltpu.SemaphoreType.DMA,
        ],
    )
    def sc_segsum(col_hbm, colf_hbm, midx_hbm, edge_hbm, sums_out, cnt_out,
                  idx_v, colf_v, midx_v, val_v, hist_v, acc_sh, cnt_sh,
                  gsem, ssem):
        cid = lax.axis_index("c")
        sid = lax.axis_index("s")
        wid = sid * _NC + cid

        def zfill(i, _):
            val_v[0, i, :] = jnp.zeros((d_edge,), jnp.float32)
            return 0
        lax.fori_loop(0, zrows, zfill, 0)
        pltpu.sync_copy(val_v.at[0, pl.ds(0, zrows)],
                        acc_sh.at[pl.ds(sid * zrows, zrows)])
        pltpu.sync_copy(val_v.at[0, pl.ds(0, czrows)],
                        cnt_sh.at[pl.ds(sid * czrows, czrows)])

        def hzero(i, _):
            hist_v[i, :] = jnp.zeros((_L,), jnp.float32)
            return 0
        lax.fori_loop(0, n_nodes // _L, hzero, 0)

        pltpu.sync_copy(col_hbm.at[wid], idx_v)
        pltpu.sync_copy(colf_hbm.at[wid], colf_v)
        pltpu.sync_copy(midx_hbm, midx_v)
        plsc.subcore_barrier()

        def gather(s, b):
            return pltpu.async_copy(
                edge_hbm.at[wid, pl.ds(s * _RS, _RS)], val_v.at[b], gsem)

        def fire_scatters(s, b):
            def body(j, _):
                pltpu.async_copy(
                    val_v.at[b, pl.ds(j * _CHUNK, _CHUNK), :],
                    acc_sh.at[idx_v.at[s * _CPS + j]], ssem, add=True)
                return 0
            lax.fori_loop(0, _CPS, body, 0)

        def drain_scatters(b):
            pltpu.make_async_copy(
                sums_out.at[cid, pl.ds(0, _RS)], val_v.at[b], ssem).wait()

        desc = [None, None]
        desc[0] = gather(0, 0)
        for s in range(nsup):
            b = s % 2
            desc[b].wait()
            if s + 1 < nsup:
                if s >= 1:
                    drain_scatters(1 - b)
                desc[1 - b] = gather(s + 1, 1 - b)
            fire_scatters(s, b)

        ones = jnp.ones((_L,), jnp.float32)
        def hbody(i, _):
            c = colf_v[i, :]
            plsc.addupdate_scatter(hist_v, [c >> 4, c & 15], ones)
            return 0
        lax.fori_loop(0, hrows, hbody, 0)

        drain_scatters(nsup % 2)
        drain_scatters((nsup - 1) % 2)

        def mbody(c, _):
            pltpu.sync_copy(hist_v.at[pl.ds(c * _CHUNK, _CHUNK)],
                            cnt_sh.at[midx_v.at[c]], add=True)
            return 0
        lax.fori_loop(0, mrg, mbody, 0)
        plsc.subcore_barrier()

        @pl.when(sid == 0)
        def _():
            pltpu.sync_copy(acc_sh, sums_out.at[cid])
            pltpu.sync_copy(cnt_sh, cnt_out.at[cid])

    return sc_segsum


def _tc_dense(x, s0, s1, c0, c1, batch2d, u, w1x, w1e, w1u, b1, gamma, beta,
              w2, b2):
    n, d_node = x.shape
    n_graphs = u.shape[0]
    d_out = w2.shape[1]

    def body(x_ref, s0_ref, s1_ref, c0_ref, c1_ref, b_ref, u_ref,
             w1x_ref, w1e_ref, w1u_ref, b1_ref, g_ref, be_ref, w2_ref, b2_ref,
             out_ref):
        sums = s0_ref[...] + s1_ref[...]
        cnt = c0_ref[...] + c1_ref[...]
        e_aggr = sums / jnp.maximum(cnt, 1.0)
        oh = (b_ref[...] == lax.broadcasted_iota(jnp.int32, (n, n_graphs), 1))
        ub = jnp.dot(oh.astype(jnp.float32), u_ref[...],
                     preferred_element_type=jnp.float32)
        h = (jnp.dot(x_ref[...], w1x_ref[...], preferred_element_type=jnp.float32)
             + jnp.dot(e_aggr, w1e_ref[...], preferred_element_type=jnp.float32)
             + jnp.dot(ub, w1u_ref[...], preferred_element_type=jnp.float32)
             + b1_ref[...])
        mean = jnp.mean(h, axis=0, keepdims=True)
        var = jnp.mean((h - mean) ** 2, axis=0, keepdims=True)
        hn = (h - mean) / jnp.sqrt(var + 1e-5) * g_ref[...] + be_ref[...]
        hr = jnp.maximum(hn, 0.0)
        out_ref[...] = (jnp.dot(hr, w2_ref[...], preferred_element_type=jnp.float32)
                        + b2_ref[...])

    return pl.pallas_call(
        body,
        out_shape=jax.ShapeDtypeStruct((n, d_out), jnp.float32),
    )(x, s0, s1, c0, c1, batch2d, u, w1x, w1e, w1u, b1, gamma, beta, w2, b2)


def kernel(x, edge_index, edge_attr, u, batch, W1, b1, gamma, beta, W2, b2):
    n_nodes, d_node = x.shape
    n_edges, d_edge = edge_attr.shape
    hidden = W1.shape[0]

    col = edge_index[1].astype(jnp.int32)
    epw = n_edges // _NW
    col3 = col.reshape(_NW, epw // _CHUNK, _CHUNK)
    colf = col.reshape(_NW, epw // _L, _L)
    edge3 = edge_attr.reshape(_NW, epw, d_edge)
    mrg = -(-(n_nodes // _L) // _CHUNK)
    midx = jnp.arange(mrg * _CHUNK, dtype=jnp.int32).reshape(mrg, _CHUNK)

    sc_segsum = _make_sc_segsum(n_nodes, n_edges, d_edge)
    sums, cnt = sc_segsum(col3, colf, midx, edge3)

    crows = cnt.shape[1]
    c0 = cnt[0].reshape(crows * _L, 1)[:n_nodes]
    c1 = cnt[1].reshape(crows * _L, 1)[:n_nodes]

    w1t = W1.T
    w1x = w1t[:d_node]
    w1e = w1t[d_node:d_node + d_edge]
    w1u = w1t[d_node + d_edge:]

    return _tc_dense(
        x, sums[0], sums[1], c0, c1,
        batch.astype(jnp.int32).reshape(n_nodes, 1), u,
        w1x, w1e, w1u,
        b1.reshape(1, hidden), gamma.reshape(1, hidden), beta.reshape(1, hidden),
        W2.T, b2.reshape(1, W2.shape[0]),
    )


# speedup vs baseline: 6.3866x; 1.0124x over previous
import functools

import jax
import jax.numpy as jnp
from jax import lax
from jax.experimental import pallas as pl
from jax.experimental.pallas import tpu as pltpu
from jax.experimental.pallas import tpu_sc as plsc

_NC = 2
_NS = 16
_NW = _NC * _NS
_CHUNK = 125
_CPS = 10
_RS = _CHUNK * _CPS
_L = 16


def _make_sc_segsum(n_nodes, n_edges, d_edge):
    epw = n_edges // _NW
    nchunk = epw // _CHUNK
    nsup = nchunk // _CPS
    zrows = n_nodes // _NS
    hrows = epw // _L
    crows = -(-(n_nodes // _L) // _NS) * _NS
    czrows = crows // _NS
    mrg = -(-(n_nodes // _L) // _CHUNK)

    mesh = plsc.VectorSubcoreMesh(core_axis_name="c", subcore_axis_name="s")

    @functools.partial(
        pl.kernel,
        out_type=(
            jax.ShapeDtypeStruct((_NC, n_nodes, d_edge), jnp.float32),
            jax.ShapeDtypeStruct((_NC, crows, _L), jnp.float32),
        ),
        mesh=mesh,
        compiler_params=pltpu.CompilerParams(use_tc_tiling_on_sc=False,
                                             needs_layout_passes=False),
        scratch_types=[
            pltpu.VMEM((nchunk, _CHUNK), jnp.int32),
            pltpu.VMEM((hrows, _L), jnp.int32),
            pltpu.VMEM((mrg, _CHUNK), jnp.int32),
            pltpu.VMEM((2, _RS, d_edge), jnp.float32),
            pltpu.VMEM((n_nodes // _L, _L), jnp.float32),
            pltpu.VMEM_SHARED((n_nodes, d_edge), jnp.float32),
            pltpu.VMEM_SHARED((crows, _L), jnp.float32),
            pltpu.SemaphoreType.DMA,
            pltpu.SemaphoreType.DMA,
        ],
    )
    def sc_segsum(col_hbm, colf_hbm, midx_hbm, edge_hbm, sums_out, cnt_out,
                  idx_v, colf_v, midx_v, val_v, hist_v, acc_sh, cnt_sh,
                  gsem, ssem):
        cid = lax.axis_index("c")
        sid = lax.axis_index("s")
        wid = sid * _NC + cid

        def zfill(i, _):
            val_v[0, i, :] = jnp.zeros((d_edge,), jnp.float32)
            return 0
        lax.fori_loop(0, zrows, zfill, 0)
        pltpu.sync_copy(val_v.at[0, pl.ds(0, zrows)],
                        acc_sh.at[pl.ds(sid * zrows, zrows)])
        pltpu.sync_copy(val_v.at[0, pl.ds(0, czrows)],
                        cnt_sh.at[pl.ds(sid * czrows, czrows)])

        def hzero(i, _):
            hist_v[i, :] = jnp.zeros((_L,), jnp.float32)
            return 0
        lax.fori_loop(0, n_nodes // _L, hzero, 0)

        pltpu.sync_copy(col_hbm.at[wid], idx_v)
        pltpu.sync_copy(colf_hbm.at[wid], colf_v)
        pltpu.sync_copy(midx_hbm, midx_v)
        plsc.subcore_barrier()

        def gather(s, b):
            return pltpu.async_copy(
                edge_hbm.at[pl.ds(wid * epw + s * _RS, _RS)], val_v.at[b], gsem)

        def fire_scatters(s, b):
            def body(j, _):
                pltpu.async_copy(
                    val_v.at[b, pl.ds(j * _CHUNK, _CHUNK), :],
                    acc_sh.at[idx_v.at[s * _CPS + j]], ssem, add=True)
                return 0
            lax.fori_loop(0, _CPS, body, 0)

        def drain_scatters(b):
            pltpu.make_async_copy(
                sums_out.at[cid, pl.ds(0, _RS)], val_v.at[b], ssem).wait()

        ones = jnp.ones((_L,), jnp.float32)
        def hbody(i, _):
            c = colf_v[i, :]
            plsc.addupdate_scatter(hist_v, [c >> 4, c & 15], ones)
            return 0
        hper = -(-hrows // nsup)

        desc = [None, None]
        desc[0] = gather(0, 0)
        for s in range(nsup):
            b = s % 2
            desc[b].wait()
            if s + 1 < nsup:
                if s >= 1:
                    drain_scatters(1 - b)
                desc[1 - b] = gather(s + 1, 1 - b)
            fire_scatters(s, b)
            lax.fori_loop(s * hper, min((s + 1) * hper, hrows), hbody, 0)

        drain_scatters(nsup % 2)
        drain_scatters((nsup - 1) % 2)

        def mbody(c, _):
            pltpu.async_copy(hist_v.at[pl.ds(c * _CHUNK, _CHUNK)],
                             cnt_sh.at[midx_v.at[c]], ssem, add=True)
            return 0
        lax.fori_loop(0, mrg, mbody, 0)
        pltpu.make_async_copy(sums_out.at[cid, pl.ds(0, hrows)],
                              hist_v, ssem).wait()
        plsc.subcore_barrier()

        @pl.when(sid == 0)
        def _():
            pltpu.sync_copy(acc_sh, sums_out.at[cid])
            pltpu.sync_copy(cnt_sh, cnt_out.at[cid])

    return sc_segsum


def _tc_dense(x, s0, s1, c0, c1, batch2d, u, w1x, w1e, w1u, b1, gamma, beta,
              w2, b2):
    n, d_node = x.shape
    n_graphs = u.shape[0]
    d_out = w2.shape[1]

    def body(x_ref, s0_ref, s1_ref, c0_ref, c1_ref, b_ref, u_ref,
             w1x_ref, w1e_ref, w1u_ref, b1_ref, g_ref, be_ref, w2_ref, b2_ref,
             out_ref):
        sums = s0_ref[...] + s1_ref[...]
        cnt = c0_ref[...] + c1_ref[...]
        e_aggr = sums / jnp.maximum(cnt, 1.0)
        oh = (b_ref[...] == lax.broadcasted_iota(jnp.int32, (n, n_graphs), 1))
        ub = jnp.dot(oh.astype(jnp.float32), u_ref[...],
                     preferred_element_type=jnp.float32)
        h = (jnp.dot(x_ref[...], w1x_ref[...], preferred_element_type=jnp.float32)
             + jnp.dot(e_aggr, w1e_ref[...], preferred_element_type=jnp.float32)
             + jnp.dot(ub, w1u_ref[...], preferred_element_type=jnp.float32)
             + b1_ref[...])
        mean = jnp.mean(h, axis=0, keepdims=True)
        var = jnp.mean((h - mean) ** 2, axis=0, keepdims=True)
        hn = (h - mean) / jnp.sqrt(var + 1e-5) * g_ref[...] + be_ref[...]
        hr = jnp.maximum(hn, 0.0)
        out_ref[...] = (jnp.dot(hr, w2_ref[...], preferred_element_type=jnp.float32)
                        + b2_ref[...])

    return pl.pallas_call(
        body,
        out_shape=jax.ShapeDtypeStruct((n, d_out), jnp.float32),
    )(x, s0, s1, c0, c1, batch2d, u, w1x, w1e, w1u, b1, gamma, beta, w2, b2)


def kernel(x, edge_index, edge_attr, u, batch, W1, b1, gamma, beta, W2, b2):
    n_nodes, d_node = x.shape
    n_edges, d_edge = edge_attr.shape
    hidden = W1.shape[0]

    col = edge_index[1].astype(jnp.int32)
    epw = n_edges // _NW
    col3 = col.reshape(_NW, epw // _CHUNK, _CHUNK)
    colf = col.reshape(_NW, epw // _L, _L)
    edge3 = edge_attr
    mrg = -(-(n_nodes // _L) // _CHUNK)
    midx = jnp.arange(mrg * _CHUNK, dtype=jnp.int32).reshape(mrg, _CHUNK)

    sc_segsum = _make_sc_segsum(n_nodes, n_edges, d_edge)
    sums, cnt = sc_segsum(col3, colf, midx, edge3)

    crows = cnt.shape[1]
    c0 = cnt[0].reshape(crows * _L, 1)[:n_nodes]
    c1 = cnt[1].reshape(crows * _L, 1)[:n_nodes]

    w1t = W1.T
    w1x = w1t[:d_node]
    w1e = w1t[d_node:d_node + d_edge]
    w1u = w1t[d_node + d_edge:]

    return _tc_dense(
        x, sums[0], sums[1], c0, c1,
        batch.astype(jnp.int32).reshape(n_nodes, 1), u,
        w1x, w1e, w1u,
        b1.reshape(1, hidden), gamma.reshape(1, hidden), beta.reshape(1, hidden),
        W2.T, b2.reshape(1, W2.shape[0]),
    )


# speedup vs baseline: 9.4510x; 1.4798x over previous
import functools

import jax
import jax.numpy as jnp
from jax import lax
from jax.experimental import pallas as pl
from jax.experimental.pallas import tpu as pltpu
from jax.experimental.pallas import tpu_sc as plsc

_NC = 2
_NS = 16
_L = 16
_BLK = 128
_CBLK = 125
_CHUNK = _CBLK * _BLK


def _make_sc_segsum(n_nodes, n_edges, d_edge):
    epc = n_edges // _NC
    nch = epc // _CHUNK
    hrows = n_nodes // _L
    crows = -(-hrows // _NS) * _NS
    czrows = crows // _NS
    mrg = -(-hrows // 125)

    mesh = plsc.VectorSubcoreMesh(core_axis_name="c", subcore_axis_name="s")

    @functools.partial(
        pl.kernel,
        out_type=(
            jax.ShapeDtypeStruct((_NC, d_edge, n_nodes // _L, _L), jnp.float32),
            jax.ShapeDtypeStruct((_NC, crows, _L), jnp.float32),
        ),
        mesh=mesh,
        compiler_params=pltpu.CompilerParams(use_tc_tiling_on_sc=False,
                                             needs_layout_passes=False),
        scratch_types=[
            pltpu.VMEM((2, _CBLK, _BLK), jnp.float32),
            pltpu.VMEM((2, _CBLK, _BLK), jnp.int32),
            pltpu.VMEM((mrg, 125), jnp.int32),
            pltpu.VMEM((n_nodes // _L, _L), jnp.float32),
            pltpu.VMEM((hrows, _L), jnp.float32),
            pltpu.VMEM_SHARED((crows, _L), jnp.float32),
            pltpu.SemaphoreType.DMA,
            pltpu.SemaphoreType.DMA,
        ],
    )
    def sc_segsum(feat_hbm, col_hbm, midx_hbm, sums_out, cnt_out,
                  val_v, colc_v, midx_v, sum_v, hist_v, cnt_sh, gsem, msem):
        cid = lax.axis_index("c")
        sid = lax.axis_index("s")
        fb = sid // 8
        fr = sid % 8

        def zfill(ref, n):
            def body(i, _):
                ref[i, :] = jnp.zeros((_L,), jnp.float32)
                return 0
            lax.fori_loop(0, n, body, 0)

        zfill(sum_v, n_nodes // _L)
        zfill(hist_v, hrows)
        pltpu.sync_copy(hist_v.at[pl.ds(0, czrows)],
                        cnt_sh.at[pl.ds(sid * czrows, czrows)])
        pltpu.sync_copy(midx_hbm, midx_v)
        plsc.subcore_barrier()

        ebase = cid * (epc // _BLK)

        def gather(k, b):
            d0 = pltpu.async_copy(
                feat_hbm.at[fb, pl.ds(ebase + k * _CBLK, _CBLK), fr],
                val_v.at[b], gsem)
            d1 = pltpu.async_copy(
                col_hbm.at[cid, pl.ds(k * _CBLK, _CBLK)], colc_v.at[b], gsem)
            return (d0, d1)

        ones = jnp.ones((_L,), jnp.float32)
        clo = (sid * _CBLK) // _NS
        chi = ((sid + 1) * _CBLK) // _NS

        def process(b):
            def body(i, _):
                for l in range(_BLK // _L):
                    c = colc_v[b, i, pl.ds(l * _L, _L)]
                    v = val_v[b, i, pl.ds(l * _L, _L)]
                    plsc.addupdate_scatter(sum_v, [c >> 4, c & 15], v)
                return 0
            lax.fori_loop(0, _CBLK, body, 0)

            def cbody(i, _):
                for l in range(_BLK // _L):
                    c = colc_v[b, i, pl.ds(l * _L, _L)]
                    plsc.addupdate_scatter(hist_v, [c >> 4, c & 15], ones)
                return 0
            lax.fori_loop(clo, chi, cbody, 0)

        desc = [None, None]
        desc[0] = gather(0, 0)
        for k in range(nch):
            b = k % 2
            desc[b][0].wait()
            desc[b][1].wait()
            if k + 1 < nch:
                desc[1 - b] = gather(k + 1, 1 - b)
            process(b)

        pltpu.sync_copy(sum_v, sums_out.at[cid, sid])

        def mbody(c, _):
            pltpu.async_copy(hist_v.at[pl.ds(c * 125, 125)],
                             cnt_sh.at[midx_v.at[c]], msem, add=True)
            return 0
        lax.fori_loop(0, mrg, mbody, 0)
        pltpu.make_async_copy(cnt_out.at[cid, pl.ds(0, hrows)],
                              hist_v, msem).wait()
        plsc.subcore_barrier()

        @pl.when(sid == 0)
        def _():
            pltpu.sync_copy(cnt_sh, cnt_out.at[cid])

    return sc_segsum


def _tc_dense(x, s0, s1, c0, c1, batch2d, u, w1x, w1e, w1u, b1, gamma, beta,
              w2, b2):
    n, d_node = x.shape
    n_graphs = u.shape[0]
    d_out = w2.shape[1]

    def body(x_ref, s0_ref, s1_ref, c0_ref, c1_ref, b_ref, u_ref,
             w1x_ref, w1e_ref, w1u_ref, b1_ref, g_ref, be_ref, w2_ref, b2_ref,
             out_ref):
        sums_t = s0_ref[...] + s1_ref[...]
        cnt = jnp.maximum(c0_ref[...] + c1_ref[...], 1.0)
        e_t = sums_t / cnt
        he = lax.dot_general(e_t, w1e_ref[...], (((0,), (0,)), ((), ())),
                             preferred_element_type=jnp.float32)
        oh = (b_ref[...] == lax.broadcasted_iota(jnp.int32, (n, n_graphs), 1))
        ub = jnp.dot(oh.astype(jnp.float32), u_ref[...],
                     preferred_element_type=jnp.float32)
        h = (jnp.dot(x_ref[...], w1x_ref[...], preferred_element_type=jnp.float32)
             + he
             + jnp.dot(ub, w1u_ref[...], preferred_element_type=jnp.float32)
             + b1_ref[...])
        mean = jnp.mean(h, axis=0, keepdims=True)
        var = jnp.mean((h - mean) ** 2, axis=0, keepdims=True)
        hn = (h - mean) / jnp.sqrt(var + 1e-5) * g_ref[...] + be_ref[...]
        hr = jnp.maximum(hn, 0.0)
        out_ref[...] = (jnp.dot(hr, w2_ref[...], preferred_element_type=jnp.float32)
                        + b2_ref[...])

    return pl.pallas_call(
        body,
        out_shape=jax.ShapeDtypeStruct((n, d_out), jnp.float32),
    )(x, s0, s1, c0, c1, batch2d, u, w1x, w1e, w1u, b1, gamma, beta, w2, b2)


def kernel(x, edge_index, edge_attr, u, batch, W1, b1, gamma, beta, W2, b2):
    n_nodes, d_node = x.shape
    n_edges, d_edge = edge_attr.shape
    hidden = W1.shape[0]

    col = edge_index[1].astype(jnp.int32)
    colc = col.reshape(_NC, n_edges // _NC // _BLK, _BLK)
    featv = edge_attr.T.reshape(d_edge // 8, 8, n_edges // _BLK, _BLK)
    featv = featv.transpose(0, 2, 1, 3)
    hrows = n_nodes // _L
    mrg = -(-hrows // 125)
    midx = jnp.arange(mrg * 125, dtype=jnp.int32).reshape(mrg, 125)

    sc_segsum = _make_sc_segsum(n_nodes, n_edges, d_edge)
    sums, cnt = sc_segsum(featv, colc, midx)
    sums = sums.reshape(_NC, d_edge, n_nodes)

    crows = cnt.shape[1]
    c0 = cnt[0].reshape(1, crows * _L)[:, :n_nodes]
    c1 = cnt[1].reshape(1, crows * _L)[:, :n_nodes]

    w1t = W1.T
    w1x = w1t[:d_node]
    w1e = w1t[d_node:d_node + d_edge]
    w1u = w1t[d_node + d_edge:]

    return _tc_dense(
        x, sums[0], sums[1], c0, c1,
        batch.astype(jnp.int32).reshape(n_nodes, 1), u,
        w1x, w1e, w1u,
        b1.reshape(1, hidden), gamma.reshape(1, hidden), beta.reshape(1, hidden),
        W2.T, b2.reshape(1, W2.shape[0]),
    )
